# Initial kernel scaffold; baseline (speedup 1.0000x reference)
import jax
import jax.numpy as jnp
from jax.experimental import pallas as pl


def kernel(x, h, edge_indices, batch_size, W1, b1, W2, b2, U1, ub1, g1, be1, U2, ub2):
    raise NotImplementedError("write your pallas kernel here")


# speedup vs baseline: 2.1508x; 2.1508x over previous
import functools

import jax
import jax.numpy as jnp
from jax import lax
from jax.experimental import pallas as pl
from jax.experimental.pallas import tpu as pltpu
from jax.experimental.pallas import tpu_sc as plsc

_NC = 2
_NS = 16
_L = 16


def _sc_dist(x0, x1, x2, row, col):
    M = x0.shape[0]
    E = row.shape[0]
    NW = _NC * _NS
    EW = E // NW
    NIT = EW // _L

    mesh = plsc.VectorSubcoreMesh(core_axis_name="c", subcore_axis_name="s")

    @functools.partial(
        pl.kernel,
        mesh=mesh,
        compiler_params=pltpu.CompilerParams(needs_layout_passes=False, use_tc_tiling_on_sc=False),
        out_type=jax.ShapeDtypeStruct((E,), jnp.float32),
        scratch_types=[
            pltpu.VMEM((M,), jnp.float32),
            pltpu.VMEM((M,), jnp.float32),
            pltpu.VMEM((M,), jnp.float32),
            pltpu.VMEM((EW,), jnp.int32),
            pltpu.VMEM((EW,), jnp.int32),
            pltpu.VMEM((EW,), jnp.float32),
        ],
    )
    def k(x0h, x1h, x2h, rowh, colh, d2h, x0v, x1v, x2v, rowv, colv, d2v):
        c = lax.axis_index("c")
        s = lax.axis_index("s")
        wid = s * _NC + c
        base = wid * EW
        pltpu.sync_copy(x0h, x0v)
        pltpu.sync_copy(x1h, x1v)
        pltpu.sync_copy(x2h, x2v)
        pltpu.sync_copy(rowh.at[pl.ds(base, EW)], rowv)
        pltpu.sync_copy(colh.at[pl.ds(base, EW)], colv)

        def body(i, carry):
            r = rowv[pl.ds(i * _L, _L)]
            q = colv[pl.ds(i * _L, _L)]
            dx = plsc.load_gather(x0v, [r]) - plsc.load_gather(x0v, [q])
            dy = plsc.load_gather(x1v, [r]) - plsc.load_gather(x1v, [q])
            dz = plsc.load_gather(x2v, [r]) - plsc.load_gather(x2v, [q])
            d2v[pl.ds(i * _L, _L)] = dx * dx + dy * dy + dz * dz
            return carry

        lax.fori_loop(0, NIT, body, 0)
        pltpu.sync_copy(d2v, d2h.at[pl.ds(base, EW)])

    return k(x0, x1, x2, row, col)


def _filter_body(d2_ref, w1_ref, b1_ref, w2_ref, b2_ref, o_ref):
    dist = jnp.sqrt(d2_ref[...])
    sh = dist * w1_ref[...] + b1_ref[...]
    sh = sh * lax.logistic(sh)
    f = jnp.dot(sh, w2_ref[...], preferred_element_type=jnp.float32)
    f = f + b2_ref[...]
    half = f.shape[-1] // 2
    o_ref[0] = f[:, :half]
    o_ref[1] = f[:, half:]


def _tc_filter(d2, W1, b1, W2, b2):
    E = d2.shape[0]
    H = W1.shape[1]
    D = W2.shape[1]
    BLK = 512
    G = E // BLK
    d2r = d2.reshape(E, 1)
    return pl.pallas_call(
        _filter_body,
        grid=(G,),
        in_specs=[
            pl.BlockSpec((BLK, 1), lambda i: (i, 0)),
            pl.BlockSpec((1, H), lambda i: (0, 0)),
            pl.BlockSpec((1, H), lambda i: (0, 0)),
            pl.BlockSpec((H, D), lambda i: (0, 0)),
            pl.BlockSpec((1, D), lambda i: (0, 0)),
        ],
        out_specs=pl.BlockSpec((2, BLK, D // 2), lambda i: (0, i, 0)),
        out_shape=jax.ShapeDtypeStruct((2, E, D // 2), jnp.float32),
    )(d2r, W1.reshape(1, H), b1.reshape(1, H), W2, b2.reshape(1, D))


def _sc_scatter(h2, f2, row, col):
    M = h2.shape[1]
    Dh = h2.shape[2]
    E = row.shape[0]
    ES = E // _NS
    CH = 80
    NCH = ES // CH
    STR = M // _NS
    NZ = Dh // _L

    mesh = plsc.VectorSubcoreMesh(core_axis_name="c", subcore_axis_name="s")

    @functools.partial(
        pl.kernel,
        mesh=mesh,
        compiler_params=pltpu.CompilerParams(needs_layout_passes=False, use_tc_tiling_on_sc=False),
        out_type=[
            jax.ShapeDtypeStruct((2, M, Dh), jnp.float32),
            jax.ShapeDtypeStruct((M, _L), jnp.float32),
        ],
        scratch_types=[
            pltpu.VMEM((CH,), jnp.int32),
            pltpu.VMEM((CH,), jnp.int32),
            pltpu.VMEM((CH, Dh), jnp.float32),
            pltpu.VMEM((CH, Dh), jnp.float32),
            pltpu.VMEM((CH, _L), jnp.float32),
            pltpu.VMEM((STR, Dh), jnp.float32),
            pltpu.VMEM_SHARED((M, Dh), jnp.float32),
            pltpu.VMEM_SHARED((M, _L), jnp.float32),
        ],
    )
    def k(h2h, f2h, rowh, colh, sums_h, cnt_h,
          rowv, colv, hrows, fv, ones, zbuf, sh_sums, sh_cnt):
        c = lax.axis_index("c")
        s = lax.axis_index("s")
        z16 = jnp.zeros((_L,), jnp.float32)
        o16 = jnp.ones((_L,), jnp.float32)

        def zrow(r, carry):
            for kk in range(NZ):
                zbuf[r, pl.ds(kk * _L, _L)] = z16
            return carry

        lax.fori_loop(0, STR, zrow, 0)

        def orow(r, carry):
            ones[r, pl.ds(0, _L)] = o16
            return carry

        lax.fori_loop(0, CH, orow, 0)

        pltpu.sync_copy(zbuf, sh_sums.at[pl.ds(s * STR, STR)])
        pltpu.sync_copy(zbuf.at[pl.ds(0, STR), pl.ds(0, _L)],
                        sh_cnt.at[pl.ds(s * STR, STR)])
        plsc.subcore_barrier()

        def chunk(i, carry):
            e0 = s * ES + i * CH
            pltpu.sync_copy(rowh.at[pl.ds(e0, CH)], rowv)
            pltpu.sync_copy(colh.at[pl.ds(e0, CH)], colv)
            pltpu.sync_copy(h2h.at[c].at[colv], hrows)
            pltpu.sync_copy(f2h.at[c, pl.ds(e0, CH)], fv)

            def mrow(r, cc):
                for kk in range(NZ):
                    sl = pl.ds(kk * _L, _L)
                    hrows[r, sl] = hrows[r, sl] * fv[r, sl]
                return cc

            lax.fori_loop(0, CH, mrow, 0)
            pltpu.sync_copy(hrows, sh_sums.at[rowv], add=True)

            @pl.when(c == 0)
            def _():
                pltpu.sync_copy(ones, sh_cnt.at[rowv], add=True)

            return carry

        lax.fori_loop(0, NCH, chunk, 0)
        plsc.subcore_barrier()
        pltpu.sync_copy(sh_sums.at[pl.ds(s * STR, STR)],
                        sums_h.at[c, pl.ds(s * STR, STR)])

        @pl.when(c == 0)
        def _():
            pltpu.sync_copy(sh_cnt.at[pl.ds(s * STR, STR)],
                            cnt_h.at[pl.ds(s * STR, STR)])

    return k(h2, f2, row, col)


def _update_body(h_ref, s_ref, c_ref, u1a, u1b0, u1b1, ub1r, g1r, be1r, u2,
                 ub2r, o_ref):
    cnt = jnp.maximum(c_ref[:, 0:1], 1.0)
    inv = 1.0 / cnt
    a0 = s_ref[0] * inv
    a1 = s_ref[1] * inv
    z = jnp.dot(h_ref[...], u1a[...], preferred_element_type=jnp.float32)
    z = z + jnp.dot(a0, u1b0[...], preferred_element_type=jnp.float32)
    z = z + jnp.dot(a1, u1b1[...], preferred_element_type=jnp.float32)
    z = z + ub1r[...]
    mu = jnp.mean(z, axis=-1, keepdims=True)
    zc = z - mu
    var = jnp.mean(zc * zc, axis=-1, keepdims=True)
    zn = zc * lax.rsqrt(var + 1e-5) * g1r[...] + be1r[...]
    zn = zn * lax.logistic(zn)
    o_ref[...] = jnp.dot(zn, u2[...], preferred_element_type=jnp.float32) \
        + ub2r[...]


def _tc_update(hf, sums2, cnt, U1a, U1b0, U1b1, ub1, g1, be1, U2, ub2):
    M, D = hf.shape
    Dh = D // 2
    H = U1a.shape[1]
    BLK = 1000
    G = M // BLK
    return pl.pallas_call(
        _update_body,
        grid=(G,),
        in_specs=[
            pl.BlockSpec((BLK, D), lambda i: (i, 0)),
            pl.BlockSpec((2, BLK, Dh), lambda i: (0, i, 0)),
            pl.BlockSpec((BLK, _L), lambda i: (i, 0)),
            pl.BlockSpec((D, H), lambda i: (0, 0)),
            pl.BlockSpec((Dh, H), lambda i: (0, 0)),
            pl.BlockSpec((Dh, H), lambda i: (0, 0)),
            pl.BlockSpec((1, H), lambda i: (0, 0)),
            pl.BlockSpec((1, H), lambda i: (0, 0)),
            pl.BlockSpec((1, H), lambda i: (0, 0)),
            pl.BlockSpec((H, D), lambda i: (0, 0)),
            pl.BlockSpec((1, D), lambda i: (0, 0)),
        ],
        out_specs=pl.BlockSpec((BLK, D), lambda i: (i, 0)),
        out_shape=jax.ShapeDtypeStruct((M, D), jnp.float32),
    )(hf, sums2, cnt, U1a, U1b0, U1b1, ub1.reshape(1, H), g1.reshape(1, H),
      be1.reshape(1, H), U2, ub2.reshape(1, D))


def kernel(x, h, edge_indices, batch_size,
           W1, b1, W2, b2, U1, ub1, g1, be1, U2, ub2):
    B, N, D = h.shape
    M = B * N
    Dh = D // 2
    row = edge_indices[0]
    col = edge_indices[1]
    xf = x.reshape(M, 3)
    hf = h.reshape(M, D)
    h2 = jnp.stack([hf[:, :Dh], hf[:, Dh:]])

    d2 = _sc_dist(xf[:, 0], xf[:, 1], xf[:, 2], row, col)
    f2 = _tc_filter(d2, W1, b1, W2, b2)
    sums2, cnt = _sc_scatter(h2, f2, row, col)
    out = _tc_update(hf, sums2, cnt,
                     U1[:D], U1[D:D + Dh], U1[D + Dh:],
                     ub1, g1, be1, U2, ub2)
    return out.reshape(B, N, D)


# speedup vs baseline: 3.2460x; 1.5092x over previous
import functools

import jax
import jax.numpy as jnp
from jax import lax
from jax.experimental import pallas as pl
from jax.experimental.pallas import tpu as pltpu
from jax.experimental.pallas import tpu_sc as plsc

_NC = 2
_NS = 16
_L = 16


def _sc_dist(x0, x1, x2, row2, col2):
    M = x0.shape[0]
    E = row2.shape[0] * row2.shape[1]
    NW = _NC * _NS
    EW = E // NW
    NCH = EW // _SUB
    NIN = _SUB // _L

    mesh = plsc.VectorSubcoreMesh(core_axis_name="c", subcore_axis_name="s")

    @functools.partial(
        pl.kernel,
        mesh=mesh,
        compiler_params=pltpu.CompilerParams(needs_layout_passes=False,
                                             use_tc_tiling_on_sc=False),
        out_type=[
            jax.ShapeDtypeStruct((E,), jnp.float32),
            jax.ShapeDtypeStruct((2, M, _L), jnp.float32),
        ],
        scratch_types=[
            pltpu.VMEM((M,), jnp.float32),
            pltpu.VMEM((M,), jnp.float32),
            pltpu.VMEM((M,), jnp.float32),
            pltpu.VMEM((EW // _SUB, _SUB), jnp.int32),
            pltpu.VMEM((EW // _SUB, _SUB), jnp.int32),
            pltpu.VMEM((EW,), jnp.float32),
            pltpu.VMEM((_SUB, _L), jnp.float32),
            pltpu.VMEM((M // _NS, _L), jnp.float32),
            pltpu.VMEM_SHARED((M, _L), jnp.float32),
            pltpu.SemaphoreType.DMA,
        ],
    )
    def k(x0h, x1h, x2h, row2h, col2h, d2h, cnt_h,
          x0v, x1v, x2v, rowc, colc, d2v, ones, zb, sh_cnt, sem_s):
        c = lax.axis_index("c")
        s = lax.axis_index("s")
        wid = s * _NC + c
        base = wid * EW
        STR = M // _NS
        pltpu.sync_copy(x0h, x0v)
        pltpu.sync_copy(x1h, x1v)
        pltpu.sync_copy(x2h, x2v)
        pltpu.sync_copy(row2h.at[pl.ds(wid * NCH, NCH)], rowc)
        pltpu.sync_copy(col2h.at[pl.ds(wid * NCH, NCH)], colc)

        o16 = jnp.ones((_L,), jnp.float32)
        z16 = jnp.zeros((_L,), jnp.float32)

        def orow(r, carry):
            ones[r, pl.ds(0, _L)] = o16
            return carry

        lax.fori_loop(0, _SUB, orow, 0)

        def zrow(r, carry):
            zb[r, pl.ds(0, _L)] = z16
            return carry

        lax.fori_loop(0, STR, zrow, 0)
        pltpu.sync_copy(zb, sh_cnt.at[pl.ds(s * STR, STR)])
        plsc.subcore_barrier()

        def cnt_desc(i):
            return pltpu.make_async_copy(ones, sh_cnt.at[rowc.at[i]], sem_s)

        def chunk(i, carry):
            for t in range(NIN):
                g = i * NIN + t
                r = rowc[i, pl.ds(t * _L, _L)]
                q = colc[i, pl.ds(t * _L, _L)]
                dx = plsc.load_gather(x0v, [r]) - plsc.load_gather(x0v, [q])
                dy = plsc.load_gather(x1v, [r]) - plsc.load_gather(x1v, [q])
                dz = plsc.load_gather(x2v, [r]) - plsc.load_gather(x2v, [q])
                d2v[pl.ds(g * _L, _L)] = dx * dx + dy * dy + dz * dz
            pltpu.async_copy(ones, sh_cnt.at[rowc.at[i]], sem_s, add=True)

            @pl.when(i >= 2)
            def _():
                cnt_desc(i - 2).wait()

            return carry

        lax.fori_loop(0, NCH, chunk, 0)
        cnt_desc(NCH - 2).wait()
        cnt_desc(NCH - 1).wait()
        pltpu.sync_copy(d2v, d2h.at[pl.ds(base, EW)])
        plsc.subcore_barrier()
        pltpu.sync_copy(sh_cnt.at[pl.ds(s * STR, STR)],
                        cnt_h.at[c, pl.ds(s * STR, STR)])

    return k(x0, x1, x2, row2, col2)


def _filter_body(d2_ref, w1_ref, b1_ref, w2_ref, b2_ref, o_ref):
    dist = jnp.sqrt(d2_ref[...])
    sh = dist * w1_ref[...] + b1_ref[...]
    sh = sh * lax.logistic(sh)
    f = jnp.dot(sh, w2_ref[...], preferred_element_type=jnp.float32)
    f = f + b2_ref[...]
    half = f.shape[-1] // 2
    o_ref[0] = f[:, :half]
    o_ref[1] = f[:, half:]


def _tc_filter(d2, W1, b1, W2, b2):
    E = d2.shape[0]
    H = W1.shape[1]
    D = W2.shape[1]
    BLK = 512
    G = E // BLK
    d2r = d2.reshape(E, 1)
    return pl.pallas_call(
        _filter_body,
        grid=(G,),
        in_specs=[
            pl.BlockSpec((BLK, 1), lambda i: (i, 0)),
            pl.BlockSpec((1, H), lambda i: (0, 0)),
            pl.BlockSpec((1, H), lambda i: (0, 0)),
            pl.BlockSpec((H, D), lambda i: (0, 0)),
            pl.BlockSpec((1, D), lambda i: (0, 0)),
        ],
        out_specs=pl.BlockSpec((2, BLK, D // 2), lambda i: (0, i, 0)),
        out_shape=jax.ShapeDtypeStruct((2, E, D // 2), jnp.float32),
    )(d2r, W1.reshape(1, H), b1.reshape(1, H), W2, b2.reshape(1, D))


_SUB = 80
_NSUB = 5
_BC = _SUB * _NSUB


def _sc_scatter(h2, f2, row2, col2):
    M = h2.shape[1]
    Dh = h2.shape[2]
    E = f2.shape[1]
    ES = E // _NS
    NBC = ES // _BC
    STR = M // _NS
    ZR = STR // 5
    NZ = Dh // _L

    mesh = plsc.VectorSubcoreMesh(core_axis_name="c", subcore_axis_name="s")

    @functools.partial(
        pl.kernel,
        mesh=mesh,
        compiler_params=pltpu.CompilerParams(needs_layout_passes=False,
                                             use_tc_tiling_on_sc=False),
        out_type=jax.ShapeDtypeStruct((2, M, Dh), jnp.float32),
        scratch_types=[
            pltpu.VMEM((_NSUB, _SUB), jnp.int32),
            pltpu.VMEM((_NSUB, _SUB), jnp.int32),
            pltpu.VMEM((_NSUB, _SUB), jnp.int32),
            pltpu.VMEM((_NSUB, _SUB), jnp.int32),
            pltpu.VMEM((_NSUB, _SUB), jnp.int32),
            pltpu.VMEM((_BC, Dh), jnp.float32),
            pltpu.VMEM((_BC, Dh), jnp.float32),
            pltpu.VMEM((_BC, Dh), jnp.float32),
            pltpu.VMEM((ZR, Dh), jnp.float32),
            pltpu.VMEM_SHARED((M, Dh), jnp.float32),
            pltpu.SemaphoreType.DMA,
            pltpu.SemaphoreType.DMA,
            pltpu.SemaphoreType.DMA,
            pltpu.SemaphoreType.DMA,
            pltpu.SemaphoreType.DMA,
        ],
    )
    def k(h2h, f2h, row2h, col2h, sums_h,
          rowb0, rowb1, colb0, colb1, rsb, fb0, fb1, hb, zb, sh_sums,
          sem_in0, sem_in1, sem_gA, sem_gB, sem_s):
        c = lax.axis_index("c")
        s = lax.axis_index("s")
        hloc = h2h.at[c]
        floc = f2h.at[c]

        def fire_in(g, rowb, colb, fb, sem):
            r0 = s * (ES // _SUB) + g * _NSUB
            e0 = s * ES + g * _BC
            pltpu.async_copy(row2h.at[pl.ds(r0, _NSUB)], rowb, sem)
            pltpu.async_copy(col2h.at[pl.ds(r0, _NSUB)], colb, sem)
            pltpu.async_copy(floc.at[pl.ds(e0, _BC)], fb, sem)

        def wait_in(rowb, colb, fb, sem):
            pltpu.make_async_copy(row2h.at[pl.ds(0, _NSUB)], rowb, sem).wait()
            pltpu.make_async_copy(col2h.at[pl.ds(0, _NSUB)], colb, sem).wait()
            pltpu.make_async_copy(floc.at[pl.ds(0, _BC)], fb, sem).wait()

        def gather_desc(colb, j, sem):
            return pltpu.make_async_copy(
                hloc.at[colb.at[j]], hb.at[pl.ds(j * _SUB, _SUB)], sem)

        def drain_scatters():
            for j in range(_NSUB):
                pltpu.make_async_copy(
                    hb.at[pl.ds(j * _SUB, _SUB)],
                    sh_sums.at[rsb.at[j]], sem_s).wait()

        def process(g, rowb, colb, fb, sem_in):
            @pl.when(g >= 1)
            def _():
                drain_scatters()

            wait_in(rowb, colb, fb, sem_in)
            gather_desc(colb, 0, sem_gA).start()
            gather_desc(colb, 1, sem_gB).start()
            for j in range(_NSUB):
                sem_g = sem_gA if j % 2 == 0 else sem_gB
                gather_desc(colb, j, sem_g).wait()
                if j + 2 < _NSUB:
                    gather_desc(colb, j + 2, sem_g).start()
                for t in range(_SUB // _L):
                    rsb[j, pl.ds(t * _L, _L)] = rowb[j, pl.ds(t * _L, _L)]

                def mrow(rr, cc):
                    r = j * _SUB + rr
                    for kk in range(NZ):
                        sl = pl.ds(kk * _L, _L)
                        hb[r, sl] = hb[r, sl] * fb[r, sl]
                    return cc

                lax.fori_loop(0, _SUB, mrow, 0)
                pltpu.async_copy(hb.at[pl.ds(j * _SUB, _SUB)],
                                 sh_sums.at[rsb.at[j]], sem_s, add=True)

            @pl.when(g + 2 < NBC)
            def _():
                fire_in(g + 2, rowb, colb, fb, sem_in)

        fire_in(0, rowb0, colb0, fb0, sem_in0)
        fire_in(1, rowb1, colb1, fb1, sem_in1)

        z16 = jnp.zeros((_L,), jnp.float32)

        def zrow(r, carry):
            for kk in range(NZ):
                zb[r, pl.ds(kk * _L, _L)] = z16
            return carry

        lax.fori_loop(0, ZR, zrow, 0)

        for t in range(STR // ZR):
            off = s * STR + t * ZR
            pltpu.sync_copy(zb, sh_sums.at[pl.ds(off, ZR)])
        plsc.subcore_barrier()

        def big(g2, carry):
            process(2 * g2, rowb0, colb0, fb0, sem_in0)
            process(2 * g2 + 1, rowb1, colb1, fb1, sem_in1)
            return carry

        lax.fori_loop(0, NBC // 2, big, 0)

        drain_scatters()
        plsc.subcore_barrier()
        pltpu.sync_copy(sh_sums.at[pl.ds(s * STR, STR)],
                        sums_h.at[c, pl.ds(s * STR, STR)])

    return k(h2, f2, row2, col2)


def _update_body(h_ref, s_ref, c_ref, u1a, u1b0, u1b1, ub1r, g1r, be1r, u2,
                 ub2r, o_ref):
    cnt = jnp.maximum(c_ref[0, :, 0:1] + c_ref[1, :, 0:1], 1.0)
    inv = 1.0 / cnt
    a0 = s_ref[0] * inv
    a1 = s_ref[1] * inv
    z = jnp.dot(h_ref[...], u1a[...], preferred_element_type=jnp.float32)
    z = z + jnp.dot(a0, u1b0[...], preferred_element_type=jnp.float32)
    z = z + jnp.dot(a1, u1b1[...], preferred_element_type=jnp.float32)
    z = z + ub1r[...]
    mu = jnp.mean(z, axis=-1, keepdims=True)
    zc = z - mu
    var = jnp.mean(zc * zc, axis=-1, keepdims=True)
    zn = zc * lax.rsqrt(var + 1e-5) * g1r[...] + be1r[...]
    zn = zn * lax.logistic(zn)
    o_ref[...] = jnp.dot(zn, u2[...], preferred_element_type=jnp.float32) \
        + ub2r[...]


def _tc_update(hf, sums2, cnt, U1a, U1b0, U1b1, ub1, g1, be1, U2, ub2):
    M, D = hf.shape
    Dh = D // 2
    H = U1a.shape[1]
    BLK = 1000
    G = M // BLK
    return pl.pallas_call(
        _update_body,
        grid=(G,),
        in_specs=[
            pl.BlockSpec((BLK, D), lambda i: (i, 0)),
            pl.BlockSpec((2, BLK, Dh), lambda i: (0, i, 0)),
            pl.BlockSpec((2, BLK, _L), lambda i: (0, i, 0)),
            pl.BlockSpec((D, H), lambda i: (0, 0)),
            pl.BlockSpec((Dh, H), lambda i: (0, 0)),
            pl.BlockSpec((Dh, H), lambda i: (0, 0)),
            pl.BlockSpec((1, H), lambda i: (0, 0)),
            pl.BlockSpec((1, H), lambda i: (0, 0)),
            pl.BlockSpec((1, H), lambda i: (0, 0)),
            pl.BlockSpec((H, D), lambda i: (0, 0)),
            pl.BlockSpec((1, D), lambda i: (0, 0)),
        ],
        out_specs=pl.BlockSpec((BLK, D), lambda i: (i, 0)),
        out_shape=jax.ShapeDtypeStruct((M, D), jnp.float32),
    )(hf, sums2, cnt, U1a, U1b0, U1b1, ub1.reshape(1, H), g1.reshape(1, H),
      be1.reshape(1, H), U2, ub2.reshape(1, D))


def kernel(x, h, edge_indices, batch_size,
           W1, b1, W2, b2, U1, ub1, g1, be1, U2, ub2):
    B, N, D = h.shape
    M = B * N
    Dh = D // 2
    row = edge_indices[0]
    col = edge_indices[1]
    xf = x.reshape(M, 3)
    hf = h.reshape(M, D)
    h2 = jnp.stack([hf[:, :Dh], hf[:, Dh:]])

    E = row.shape[0]
    row2 = row.reshape(E // _SUB, _SUB)
    col2 = col.reshape(E // _SUB, _SUB)
    d2, cnt2 = _sc_dist(xf[:, 0], xf[:, 1], xf[:, 2], row2, col2)
    f2 = _tc_filter(d2, W1, b1, W2, b2)
    sums2 = _sc_scatter(h2, f2, row2, col2)
    out = _tc_update(hf, sums2, cnt2,
                     U1[:D], U1[D:D + Dh], U1[D + Dh:],
                     ub1, g1, be1, U2, ub2)
    return out.reshape(B, N, D)


# speedup vs baseline: 9.4542x; 2.9126x over previous
import functools

import jax
import jax.numpy as jnp
from jax import lax
from jax.experimental import pallas as pl
from jax.experimental.pallas import tpu as pltpu
from jax.experimental.pallas import tpu_sc as plsc

_NC = 2
_NS = 16
_L = 16


def _sc_dist(x0, x1, x2, row2, col2):
    M = x0.shape[0]
    E = row2.shape[0] * row2.shape[1]
    NW = _NC * _NS
    EW = E // NW
    NCH = EW // _SUB
    NIN = _SUB // _L

    mesh = plsc.VectorSubcoreMesh(core_axis_name="c", subcore_axis_name="s")

    @functools.partial(
        pl.kernel,
        mesh=mesh,
        compiler_params=pltpu.CompilerParams(needs_layout_passes=False,
                                             use_tc_tiling_on_sc=False),
        out_type=[
            jax.ShapeDtypeStruct((E,), jnp.float32),
            jax.ShapeDtypeStruct((2, M, _L), jnp.float32),
        ],
        scratch_types=[
            pltpu.VMEM((M,), jnp.float32),
            pltpu.VMEM((M,), jnp.float32),
            pltpu.VMEM((M,), jnp.float32),
            pltpu.VMEM((EW // _SUB, _SUB), jnp.int32),
            pltpu.VMEM((EW // _SUB, _SUB), jnp.int32),
            pltpu.VMEM((EW,), jnp.float32),
            pltpu.VMEM((_SUB, _L), jnp.float32),
            pltpu.VMEM((M // _NS, _L), jnp.float32),
            pltpu.VMEM_SHARED((M, _L), jnp.float32),
            pltpu.SemaphoreType.DMA,
        ],
    )
    def k(x0h, x1h, x2h, row2h, col2h, d2h, cnt_h,
          x0v, x1v, x2v, rowc, colc, d2v, ones, zb, sh_cnt, sem_s):
        c = lax.axis_index("c")
        s = lax.axis_index("s")
        wid = s * _NC + c
        base = wid * EW
        STR = M // _NS
        pltpu.sync_copy(x0h, x0v)
        pltpu.sync_copy(x1h, x1v)
        pltpu.sync_copy(x2h, x2v)
        pltpu.sync_copy(row2h.at[pl.ds(wid * NCH, NCH)], rowc)
        pltpu.sync_copy(col2h.at[pl.ds(wid * NCH, NCH)], colc)

        o16 = jnp.ones((_L,), jnp.float32)
        z16 = jnp.zeros((_L,), jnp.float32)

        def orow(r, carry):
            ones[r, pl.ds(0, _L)] = o16
            return carry

        lax.fori_loop(0, _SUB, orow, 0)

        def zrow(r, carry):
            zb[r, pl.ds(0, _L)] = z16
            return carry

        lax.fori_loop(0, STR, zrow, 0)
        pltpu.sync_copy(zb, sh_cnt.at[pl.ds(s * STR, STR)])
        plsc.subcore_barrier()

        def cnt_desc(i):
            return pltpu.make_async_copy(ones, sh_cnt.at[rowc.at[i]], sem_s)

        def chunk(i, carry):
            for t in range(NIN):
                g = i * NIN + t
                r = rowc[i, pl.ds(t * _L, _L)]
                q = colc[i, pl.ds(t * _L, _L)]
                dx = plsc.load_gather(x0v, [r]) - plsc.load_gather(x0v, [q])
                dy = plsc.load_gather(x1v, [r]) - plsc.load_gather(x1v, [q])
                dz = plsc.load_gather(x2v, [r]) - plsc.load_gather(x2v, [q])
                d2v[pl.ds(g * _L, _L)] = dx * dx + dy * dy + dz * dz
            pltpu.async_copy(ones, sh_cnt.at[rowc.at[i]], sem_s, add=True)

            @pl.when(i >= 2)
            def _():
                cnt_desc(i - 2).wait()

            return carry

        lax.fori_loop(0, NCH, chunk, 0)
        cnt_desc(NCH - 2).wait()
        cnt_desc(NCH - 1).wait()
        pltpu.sync_copy(d2v, d2h.at[pl.ds(base, EW)])
        plsc.subcore_barrier()
        pltpu.sync_copy(sh_cnt.at[pl.ds(s * STR, STR)],
                        cnt_h.at[c, pl.ds(s * STR, STR)])

    return k(x0, x1, x2, row2, col2)


_RB = 32


def _filter_body(d2_ref, w1c_ref, b1c_ref, w2_ref, b2_ref, o_ref):
    dist = jnp.sqrt(d2_ref[...])
    w1c = w1c_ref[...]
    b1c = b1c_ref[...]
    parts = [w1c * dist[r:r + 1, :] + b1c for r in range(_RB)]
    st = jnp.concatenate(parts, axis=1)
    st = st * lax.logistic(st)
    f = lax.dot_general(st, w2_ref[...], (((0,), (0,)), ((), ())),
                        preferred_element_type=jnp.float32)
    o_ref[...] = f + b2_ref[...]


def _tc_filter(d2m, W1, b1, W2, b2):
    R = d2m.shape[0]
    H = W1.shape[1]
    D = W2.shape[1]
    G = R // _RB
    BLK = _RB * 128
    return pl.pallas_call(
        _filter_body,
        grid=(G,),
        in_specs=[
            pl.BlockSpec((_RB, 128), lambda i: (i, 0)),
            pl.BlockSpec((H, 1), lambda i: (0, 0)),
            pl.BlockSpec((H, 1), lambda i: (0, 0)),
            pl.BlockSpec((H, D), lambda i: (0, 0)),
            pl.BlockSpec((1, D), lambda i: (0, 0)),
        ],
        out_specs=pl.BlockSpec((BLK, D), lambda i: (i, 0)),
        out_shape=jax.ShapeDtypeStruct((R * 128, D), jnp.float32),
    )(d2m, W1.reshape(H, 1), b1.reshape(H, 1), W2, b2.reshape(1, D))


_SUB = 80
_NSUB = 5
_BC = _SUB * _NSUB


def _sc_scatter(h2, f2, row2, col2):
    M = h2.shape[1]
    Dh = h2.shape[2]
    E = row2.shape[0] * row2.shape[1]
    ES = E // _NS
    NBC = ES // _BC
    STR = M // _NS
    ZR = STR // 5
    NZ = Dh // _L

    mesh = plsc.VectorSubcoreMesh(core_axis_name="c", subcore_axis_name="s")

    @functools.partial(
        pl.kernel,
        mesh=mesh,
        compiler_params=pltpu.CompilerParams(needs_layout_passes=False,
                                             use_tc_tiling_on_sc=False),
        out_type=jax.ShapeDtypeStruct((M, 2 * Dh), jnp.float32),
        scratch_types=[
            pltpu.VMEM((_NSUB, _SUB), jnp.int32),
            pltpu.VMEM((_NSUB, _SUB), jnp.int32),
            pltpu.VMEM((_NSUB, _SUB), jnp.int32),
            pltpu.VMEM((_NSUB, _SUB), jnp.int32),
            pltpu.VMEM((_NSUB, _SUB), jnp.int32),
            pltpu.VMEM((_BC, Dh), jnp.float32),
            pltpu.VMEM((_BC, Dh), jnp.float32),
            pltpu.VMEM((_BC, Dh), jnp.float32),
            pltpu.VMEM((ZR, Dh), jnp.float32),
            pltpu.VMEM_SHARED((M, Dh), jnp.float32),
            pltpu.SemaphoreType.DMA,
            pltpu.SemaphoreType.DMA,
            pltpu.SemaphoreType.DMA,
            pltpu.SemaphoreType.DMA,
            pltpu.SemaphoreType.DMA,
        ],
    )
    def k(h2h, f2h, row2h, col2h, sums_h,
          rowb0, rowb1, colb0, colb1, rsb, fb0, fb1, hb, zb, sh_sums,
          sem_in0, sem_in1, sem_gA, sem_gB, sem_s):
        c = lax.axis_index("c")
        s = lax.axis_index("s")
        hloc = h2h.at[c]
        cof = c * Dh

        def fire_in(g, rowb, colb, fb, sem):
            r0 = s * (ES // _SUB) + g * _NSUB
            e0 = s * ES + g * _BC
            pltpu.async_copy(row2h.at[pl.ds(r0, _NSUB)], rowb, sem)
            pltpu.async_copy(col2h.at[pl.ds(r0, _NSUB)], colb, sem)
            pltpu.async_copy(f2h.at[pl.ds(e0, _BC), pl.ds(cof, Dh)], fb, sem)

        def wait_in(rowb, colb, fb, sem):
            pltpu.make_async_copy(row2h.at[pl.ds(0, _NSUB)], rowb, sem).wait()
            pltpu.make_async_copy(col2h.at[pl.ds(0, _NSUB)], colb, sem).wait()
            pltpu.make_async_copy(f2h.at[pl.ds(0, _BC), pl.ds(cof, Dh)],
                                  fb, sem).wait()

        def gather_desc(colb, j, sem):
            return pltpu.make_async_copy(
                hloc.at[colb.at[j]], hb.at[pl.ds(j * _SUB, _SUB)], sem)

        def drain_scatters():
            for j in range(_NSUB):
                pltpu.make_async_copy(
                    hb.at[pl.ds(j * _SUB, _SUB)],
                    sh_sums.at[rsb.at[j]], sem_s).wait()

        def process(g, rowb, colb, fb, sem_in):
            @pl.when(g >= 1)
            def _():
                drain_scatters()

            wait_in(rowb, colb, fb, sem_in)
            gather_desc(colb, 0, sem_gA).start()
            gather_desc(colb, 1, sem_gB).start()
            for j in range(_NSUB):
                sem_g = sem_gA if j % 2 == 0 else sem_gB
                gather_desc(colb, j, sem_g).wait()
                if j + 2 < _NSUB:
                    gather_desc(colb, j + 2, sem_g).start()
                for t in range(_SUB // _L):
                    rsb[j, pl.ds(t * _L, _L)] = rowb[j, pl.ds(t * _L, _L)]

                def mrow(rr, cc):
                    r = j * _SUB + rr
                    for kk in range(NZ):
                        sl = pl.ds(kk * _L, _L)
                        hb[r, sl] = hb[r, sl] * fb[r, sl]
                    return cc

                lax.fori_loop(0, _SUB, mrow, 0)
                pltpu.async_copy(hb.at[pl.ds(j * _SUB, _SUB)],
                                 sh_sums.at[rsb.at[j]], sem_s, add=True)

            @pl.when(g + 2 < NBC)
            def _():
                fire_in(g + 2, rowb, colb, fb, sem_in)

        fire_in(0, rowb0, colb0, fb0, sem_in0)
        fire_in(1, rowb1, colb1, fb1, sem_in1)

        z16 = jnp.zeros((_L,), jnp.float32)

        def zrow(r, carry):
            for kk in range(NZ):
                zb[r, pl.ds(kk * _L, _L)] = z16
            return carry

        lax.fori_loop(0, ZR, zrow, 0)

        for t in range(STR // ZR):
            off = s * STR + t * ZR
            pltpu.sync_copy(zb, sh_sums.at[pl.ds(off, ZR)])
        plsc.subcore_barrier()

        def big(g2, carry):
            process(2 * g2, rowb0, colb0, fb0, sem_in0)
            process(2 * g2 + 1, rowb1, colb1, fb1, sem_in1)
            return carry

        lax.fori_loop(0, NBC // 2, big, 0)

        drain_scatters()
        plsc.subcore_barrier()
        pltpu.sync_copy(sh_sums.at[pl.ds(s * STR, STR)],
                        sums_h.at[pl.ds(s * STR, STR), pl.ds(cof, Dh)])

    return k(h2, f2, row2, col2)


def _update_body(h_ref, s_ref, c_ref, u1a, u1b, ub1r, g1r, be1r, u2,
                 ub2r, o_ref):
    cnt = jnp.maximum(c_ref[0, :, 0:1] + c_ref[1, :, 0:1], 1.0)
    inv = 1.0 / cnt
    agg = s_ref[...] * inv
    z = jnp.dot(h_ref[...], u1a[...], preferred_element_type=jnp.float32)
    z = z + jnp.dot(agg, u1b[...], preferred_element_type=jnp.float32)
    z = z + ub1r[...]
    mu = jnp.mean(z, axis=-1, keepdims=True)
    zc = z - mu
    var = jnp.mean(zc * zc, axis=-1, keepdims=True)
    zn = zc * lax.rsqrt(var + 1e-5) * g1r[...] + be1r[...]
    zn = zn * lax.logistic(zn)
    o_ref[...] = jnp.dot(zn, u2[...], preferred_element_type=jnp.float32) \
        + ub2r[...]


def _tc_update(hf, sums, cnt, U1a, U1b, ub1, g1, be1, U2, ub2):
    M, D = hf.shape
    H = U1a.shape[1]
    BLK = 1000
    G = M // BLK
    return pl.pallas_call(
        _update_body,
        grid=(G,),
        in_specs=[
            pl.BlockSpec((BLK, D), lambda i: (i, 0)),
            pl.BlockSpec((BLK, D), lambda i: (i, 0)),
            pl.BlockSpec((2, BLK, _L), lambda i: (0, i, 0)),
            pl.BlockSpec((D, H), lambda i: (0, 0)),
            pl.BlockSpec((D, H), lambda i: (0, 0)),
            pl.BlockSpec((1, H), lambda i: (0, 0)),
            pl.BlockSpec((1, H), lambda i: (0, 0)),
            pl.BlockSpec((1, H), lambda i: (0, 0)),
            pl.BlockSpec((H, D), lambda i: (0, 0)),
            pl.BlockSpec((1, D), lambda i: (0, 0)),
        ],
        out_specs=pl.BlockSpec((BLK, D), lambda i: (i, 0)),
        out_shape=jax.ShapeDtypeStruct((M, D), jnp.float32),
    )(hf, sums, cnt, U1a, U1b, ub1.reshape(1, H), g1.reshape(1, H),
      be1.reshape(1, H), U2, ub2.reshape(1, D))


def kernel(x, h, edge_indices, batch_size,
           W1, b1, W2, b2, U1, ub1, g1, be1, U2, ub2):
    B, N, D = h.shape
    M = B * N
    Dh = D // 2
    row = edge_indices[0]
    col = edge_indices[1]
    xf = x.reshape(M, 3)
    hf = h.reshape(M, D)
    h2 = jnp.stack([hf[:, :Dh], hf[:, Dh:]])

    E = row.shape[0]
    row2 = row.reshape(E // _SUB, _SUB)
    col2 = col.reshape(E // _SUB, _SUB)
    d2, cnt2 = _sc_dist(xf[:, 0], xf[:, 1], xf[:, 2], row2, col2)
    EP = ((E + _RB * 128 - 1) // (_RB * 128)) * (_RB * 128)
    d2m = jnp.concatenate([d2, jnp.zeros((EP - E,), jnp.float32)])
    d2m = d2m.reshape(EP // 128, 128)
    f2 = _tc_filter(d2m, W1, b1, W2, b2)
    sums = _sc_scatter(h2, f2, row2, col2)
    out = _tc_update(hf, sums, cnt2, U1[:D], U1[D:], ub1, g1, be1, U2, ub2)
    return out.reshape(B, N, D)


# speedup vs baseline: 9.9276x; 1.0501x over previous
import functools

import jax
import jax.numpy as jnp
from jax import lax
from jax.experimental import pallas as pl
from jax.experimental.pallas import tpu as pltpu
from jax.experimental.pallas import tpu_sc as plsc

_NC = 2
_NS = 16
_L = 16


def _sc_dist(x0, x1, x2, row2, col2):
    M = x0.shape[0]
    E = row2.shape[0] * row2.shape[1]
    NW = _NC * _NS
    EW = E // NW
    NCH = EW // _SUB
    NIN = _SUB // _L

    mesh = plsc.VectorSubcoreMesh(core_axis_name="c", subcore_axis_name="s")

    @functools.partial(
        pl.kernel,
        mesh=mesh,
        compiler_params=pltpu.CompilerParams(needs_layout_passes=False,
                                             use_tc_tiling_on_sc=False),
        out_type=[
            jax.ShapeDtypeStruct((E,), jnp.float32),
            jax.ShapeDtypeStruct((2, M, _L), jnp.float32),
        ],
        scratch_types=[
            pltpu.VMEM((M,), jnp.float32),
            pltpu.VMEM((M,), jnp.float32),
            pltpu.VMEM((M,), jnp.float32),
            pltpu.VMEM((EW // _SUB, _SUB), jnp.int32),
            pltpu.VMEM((EW // _SUB, _SUB), jnp.int32),
            pltpu.VMEM((EW,), jnp.float32),
            pltpu.VMEM((_SUB, _L), jnp.float32),
            pltpu.VMEM((M // _NS, _L), jnp.float32),
            pltpu.VMEM_SHARED((M, _L), jnp.float32),
            pltpu.SemaphoreType.DMA,
        ],
    )
    def k(x0h, x1h, x2h, row2h, col2h, d2h, cnt_h,
          x0v, x1v, x2v, rowc, colc, d2v, ones, zb, sh_cnt, sem_s):
        c = lax.axis_index("c")
        s = lax.axis_index("s")
        wid = s * _NC + c
        base = wid * EW
        STR = M // _NS
        pltpu.sync_copy(x0h, x0v)
        pltpu.sync_copy(x1h, x1v)
        pltpu.sync_copy(x2h, x2v)
        pltpu.sync_copy(row2h.at[pl.ds(wid * NCH, NCH)], rowc)
        pltpu.sync_copy(col2h.at[pl.ds(wid * NCH, NCH)], colc)

        o16 = jnp.ones((_L,), jnp.float32)
        z16 = jnp.zeros((_L,), jnp.float32)

        def orow(r, carry):
            ones[r, pl.ds(0, _L)] = o16
            return carry

        lax.fori_loop(0, _SUB, orow, 0)

        def zrow(r, carry):
            zb[r, pl.ds(0, _L)] = z16
            return carry

        lax.fori_loop(0, STR, zrow, 0)
        pltpu.sync_copy(zb, sh_cnt.at[pl.ds(s * STR, STR)])
        plsc.subcore_barrier()

        def cnt_desc(i):
            return pltpu.make_async_copy(ones, sh_cnt.at[rowc.at[i]], sem_s)

        def chunk(i, carry):
            for t in range(NIN):
                g = i * NIN + t
                r = rowc[i, pl.ds(t * _L, _L)]
                q = colc[i, pl.ds(t * _L, _L)]
                dx = plsc.load_gather(x0v, [r]) - plsc.load_gather(x0v, [q])
                dy = plsc.load_gather(x1v, [r]) - plsc.load_gather(x1v, [q])
                dz = plsc.load_gather(x2v, [r]) - plsc.load_gather(x2v, [q])
                d2v[pl.ds(g * _L, _L)] = dx * dx + dy * dy + dz * dz
            pltpu.async_copy(ones, sh_cnt.at[rowc.at[i]], sem_s, add=True)

            @pl.when(i >= 2)
            def _():
                cnt_desc(i - 2).wait()

            return carry

        lax.fori_loop(0, NCH, chunk, 0)
        cnt_desc(NCH - 2).wait()
        cnt_desc(NCH - 1).wait()
        pltpu.sync_copy(d2v, d2h.at[pl.ds(base, EW)])
        plsc.subcore_barrier()
        pltpu.sync_copy(sh_cnt.at[pl.ds(s * STR, STR)],
                        cnt_h.at[c, pl.ds(s * STR, STR)])

    return k(x0, x1, x2, row2, col2)


_RB = 32


def _filter_body(d2_ref, w1c_ref, b1c_ref, w2_ref, b2_ref, o_ref):
    dist = jnp.sqrt(d2_ref[...])
    w1c = w1c_ref[...]
    b1c = b1c_ref[...]
    parts = [w1c * dist[r:r + 1, :] + b1c for r in range(_RB)]
    st = jnp.concatenate(parts, axis=1)
    st = st * lax.logistic(st)
    f = lax.dot_general(st, w2_ref[...], (((0,), (0,)), ((), ())),
                        preferred_element_type=jnp.float32)
    o_ref[...] = f + b2_ref[...]


def _tc_filter(d2m, W1, b1, W2, b2):
    R = d2m.shape[0]
    H = W1.shape[1]
    D = W2.shape[1]
    G = R // _RB
    BLK = _RB * 128
    return pl.pallas_call(
        _filter_body,
        grid=(G,),
        in_specs=[
            pl.BlockSpec((_RB, 128), lambda i: (i, 0)),
            pl.BlockSpec((H, 1), lambda i: (0, 0)),
            pl.BlockSpec((H, 1), lambda i: (0, 0)),
            pl.BlockSpec((H, D), lambda i: (0, 0)),
            pl.BlockSpec((1, D), lambda i: (0, 0)),
        ],
        out_specs=pl.BlockSpec((BLK, D), lambda i: (i, 0)),
        out_shape=jax.ShapeDtypeStruct((R * 128, D), jnp.float32),
    )(d2m, W1.reshape(H, 1), b1.reshape(H, 1), W2, b2.reshape(1, D))


_SUB = 80
_NSUB = 5
_BC = _SUB * _NSUB


def _sc_scatter(h2, f2, row2, col2):
    M = h2.shape[1]
    Dh = h2.shape[2]
    E = row2.shape[0] * row2.shape[1]
    ES = E // _NS
    NBC = ES // _BC
    STR = M // _NS
    ZR = STR // 5
    NZ = Dh // _L

    mesh = plsc.VectorSubcoreMesh(core_axis_name="c", subcore_axis_name="s")

    @functools.partial(
        pl.kernel,
        mesh=mesh,
        compiler_params=pltpu.CompilerParams(needs_layout_passes=False,
                                             use_tc_tiling_on_sc=False),
        out_type=jax.ShapeDtypeStruct((M, 2 * Dh), jnp.float32),
        scratch_types=[
            pltpu.VMEM((_NSUB, _SUB), jnp.int32),
            pltpu.VMEM((_NSUB, _SUB), jnp.int32),
            pltpu.VMEM((_NSUB, _SUB), jnp.int32),
            pltpu.VMEM((_NSUB, _SUB), jnp.int32),
            pltpu.VMEM((_NSUB, _SUB), jnp.int32),
            pltpu.VMEM((_BC, Dh), jnp.float32),
            pltpu.VMEM((_BC, Dh), jnp.float32),
            pltpu.VMEM((_BC, Dh), jnp.float32),
            pltpu.VMEM((ZR, Dh), jnp.float32),
            pltpu.VMEM_SHARED((M, Dh), jnp.float32),
            pltpu.SemaphoreType.DMA,
            pltpu.SemaphoreType.DMA,
            pltpu.SemaphoreType.DMA,
            pltpu.SemaphoreType.DMA,
            pltpu.SemaphoreType.DMA,
        ],
    )
    def k(h2h, f2h, row2h, col2h, sums_h,
          rowb0, rowb1, colb0, colb1, rsb, fb0, fb1, hb, zb, sh_sums,
          sem_in0, sem_in1, sem_gA, sem_gB, sem_s):
        c = lax.axis_index("c")
        s = lax.axis_index("s")
        hloc = h2h.at[c]
        cof = c * Dh

        def fire_in(g, rowb, colb, fb, sem):
            r0 = s * (ES // _SUB) + g * _NSUB
            e0 = s * ES + g * _BC
            pltpu.async_copy(row2h.at[pl.ds(r0, _NSUB)], rowb, sem)
            pltpu.async_copy(col2h.at[pl.ds(r0, _NSUB)], colb, sem)
            pltpu.async_copy(f2h.at[pl.ds(e0, _BC), pl.ds(cof, Dh)], fb, sem)

        def wait_in(rowb, colb, fb, sem):
            pltpu.make_async_copy(row2h.at[pl.ds(0, _NSUB)], rowb, sem).wait()
            pltpu.make_async_copy(col2h.at[pl.ds(0, _NSUB)], colb, sem).wait()
            pltpu.make_async_copy(f2h.at[pl.ds(0, _BC), pl.ds(cof, Dh)],
                                  fb, sem).wait()

        def gather_desc(colb, j, sem):
            return pltpu.make_async_copy(
                hloc.at[colb.at[j]], hb.at[pl.ds(j * _SUB, _SUB)], sem)

        def drain_scatters():
            for j in range(_NSUB):
                pltpu.make_async_copy(
                    hb.at[pl.ds(j * _SUB, _SUB)],
                    sh_sums.at[rsb.at[j]], sem_s).wait()

        def process(g, rowb, colb, fb, sem_in):
            @pl.when(g >= 1)
            def _():
                drain_scatters()

            wait_in(rowb, colb, fb, sem_in)
            gather_desc(colb, 0, sem_gA).start()
            gather_desc(colb, 1, sem_gB).start()
            for j in range(_NSUB):
                sem_g = sem_gA if j % 2 == 0 else sem_gB
                gather_desc(colb, j, sem_g).wait()
                if j + 2 < _NSUB:
                    gather_desc(colb, j + 2, sem_g).start()
                for t in range(_SUB // _L):
                    rsb[j, pl.ds(t * _L, _L)] = rowb[j, pl.ds(t * _L, _L)]

                def mrow(rr, cc):
                    r = j * _SUB + rr
                    for kk in range(NZ):
                        sl = pl.ds(kk * _L, _L)
                        hb[r, sl] = hb[r, sl] * fb[r, sl]
                    return cc

                lax.fori_loop(0, _SUB, mrow, 0)
                pltpu.async_copy(hb.at[pl.ds(j * _SUB, _SUB)],
                                 sh_sums.at[rsb.at[j]], sem_s, add=True)

            @pl.when(g + 2 < NBC)
            def _():
                fire_in(g + 2, rowb, colb, fb, sem_in)

        fire_in(0, rowb0, colb0, fb0, sem_in0)
        fire_in(1, rowb1, colb1, fb1, sem_in1)

        z16 = jnp.zeros((_L,), jnp.float32)

        def zrow(r, carry):
            for kk in range(NZ):
                zb[r, pl.ds(kk * _L, _L)] = z16
            return carry

        lax.fori_loop(0, ZR, zrow, 0)

        for t in range(STR // ZR):
            off = s * STR + t * ZR
            pltpu.sync_copy(zb, sh_sums.at[pl.ds(off, ZR)])
        plsc.subcore_barrier()

        def big(g2, carry):
            process(2 * g2, rowb0, colb0, fb0, sem_in0)
            process(2 * g2 + 1, rowb1, colb1, fb1, sem_in1)
            return carry

        lax.fori_loop(0, NBC // 2, big, 0)
        if NBC % 2:
            process(NBC - 1, rowb0, colb0, fb0, sem_in0)

        drain_scatters()
        plsc.subcore_barrier()
        pltpu.sync_copy(sh_sums.at[pl.ds(s * STR, STR)],
                        sums_h.at[pl.ds(s * STR, STR), pl.ds(cof, Dh)])

    return k(h2, f2, row2, col2)


def _update_body(h_ref, s_ref, s1_ref, c_ref, u1a, u1b, ub1r, g1r, be1r, u2,
                 ub2r, o_ref):
    cnt = jnp.maximum(c_ref[0, :, 0:1] + c_ref[1, :, 0:1], 1.0)
    inv = 1.0 / cnt
    agg = (s_ref[...] + s1_ref[...]) * inv
    z = jnp.dot(h_ref[...], u1a[...], preferred_element_type=jnp.float32)
    z = z + jnp.dot(agg, u1b[...], preferred_element_type=jnp.float32)
    z = z + ub1r[...]
    mu = jnp.mean(z, axis=-1, keepdims=True)
    zc = z - mu
    var = jnp.mean(zc * zc, axis=-1, keepdims=True)
    zn = zc * lax.rsqrt(var + 1e-5) * g1r[...] + be1r[...]
    zn = zn * lax.logistic(zn)
    o_ref[...] = jnp.dot(zn, u2[...], preferred_element_type=jnp.float32) \
        + ub2r[...]


def _tc_update(hf, sums, sums1, cnt, U1a, U1b, ub1, g1, be1, U2, ub2):
    M, D = hf.shape
    H = U1a.shape[1]
    BLK = 1000
    G = M // BLK
    return pl.pallas_call(
        _update_body,
        grid=(G,),
        in_specs=[
            pl.BlockSpec((BLK, D), lambda i: (i, 0)),
            pl.BlockSpec((BLK, D), lambda i: (i, 0)),
            pl.BlockSpec((BLK, D), lambda i: (i, 0)),
            pl.BlockSpec((2, BLK, _L), lambda i: (0, i, 0)),
            pl.BlockSpec((D, H), lambda i: (0, 0)),
            pl.BlockSpec((D, H), lambda i: (0, 0)),
            pl.BlockSpec((1, H), lambda i: (0, 0)),
            pl.BlockSpec((1, H), lambda i: (0, 0)),
            pl.BlockSpec((1, H), lambda i: (0, 0)),
            pl.BlockSpec((H, D), lambda i: (0, 0)),
            pl.BlockSpec((1, D), lambda i: (0, 0)),
        ],
        out_specs=pl.BlockSpec((BLK, D), lambda i: (i, 0)),
        out_shape=jax.ShapeDtypeStruct((M, D), jnp.float32),
    )(hf, sums, sums1, cnt, U1a, U1b, ub1.reshape(1, H), g1.reshape(1, H),
      be1.reshape(1, H), U2, ub2.reshape(1, D))


def kernel(x, h, edge_indices, batch_size,
           W1, b1, W2, b2, U1, ub1, g1, be1, U2, ub2):
    B, N, D = h.shape
    M = B * N
    Dh = D // 2
    row = edge_indices[0]
    col = edge_indices[1]
    xf = x.reshape(M, 3)
    hf = h.reshape(M, D)
    h2 = jnp.stack([hf[:, :Dh], hf[:, Dh:]])

    E = row.shape[0]
    row2 = row.reshape(E // _SUB, _SUB)
    col2 = col.reshape(E // _SUB, _SUB)
    d2, cnt2 = _sc_dist(xf[:, 0], xf[:, 1], xf[:, 2], row2, col2)
    EH = E // 2
    RH = E // _SUB // 2
    BLKE = _RB * 128
    EHP = ((EH + BLKE - 1) // BLKE) * BLKE
    pad = jnp.zeros((EHP - EH,), jnp.float32)
    parts = []
    for k in range(2):
        d2k = d2[k * EH:(k + 1) * EH]
        d2mk = jnp.concatenate([d2k, pad]).reshape(EHP // 128, 128)
        f2k = _tc_filter(d2mk, W1, b1, W2, b2)
        parts.append(_sc_scatter(h2, f2k, row2[k * RH:(k + 1) * RH],
                                 col2[k * RH:(k + 1) * RH]))
    out = _tc_update(hf, parts[0], parts[1], cnt2, U1[:D], U1[D:],
                     ub1, g1, be1, U2, ub2)
    return out.reshape(B, N, D)


# speedup vs baseline: 10.1402x; 1.0214x over previous
import functools

import jax
import jax.numpy as jnp
from jax import lax
from jax.experimental import pallas as pl
from jax.experimental.pallas import tpu as pltpu
from jax.experimental.pallas import tpu_sc as plsc

_NC = 2
_NS = 16
_L = 16


def _sc_dist(x0, x1, x2, row2, col2):
    M = x0.shape[0]
    E = row2.shape[0] * row2.shape[1]
    NW = _NC * _NS
    EW = E // NW
    NCH = EW // _SUB
    NIN = _SUB // _L

    mesh = plsc.VectorSubcoreMesh(core_axis_name="c", subcore_axis_name="s")

    @functools.partial(
        pl.kernel,
        mesh=mesh,
        compiler_params=pltpu.CompilerParams(needs_layout_passes=False,
                                             use_tc_tiling_on_sc=False),
        out_type=[
            jax.ShapeDtypeStruct((E,), jnp.float32),
            jax.ShapeDtypeStruct((2, M, _L), jnp.float32),
        ],
        scratch_types=[
            pltpu.VMEM((M,), jnp.float32),
            pltpu.VMEM((M,), jnp.float32),
            pltpu.VMEM((M,), jnp.float32),
            pltpu.VMEM((EW // _SUB, _SUB), jnp.int32),
            pltpu.VMEM((EW // _SUB, _SUB), jnp.int32),
            pltpu.VMEM((EW,), jnp.float32),
            pltpu.VMEM((_SUB, _L), jnp.float32),
            pltpu.VMEM((M // _NS, _L), jnp.float32),
            pltpu.VMEM_SHARED((M, _L), jnp.float32),
            pltpu.SemaphoreType.DMA,
        ],
    )
    def k(x0h, x1h, x2h, row2h, col2h, d2h, cnt_h,
          x0v, x1v, x2v, rowc, colc, d2v, ones, zb, sh_cnt, sem_s):
        c = lax.axis_index("c")
        s = lax.axis_index("s")
        wid = s * _NC + c
        base = wid * EW
        STR = M // _NS
        pltpu.sync_copy(x0h, x0v)
        pltpu.sync_copy(x1h, x1v)
        pltpu.sync_copy(x2h, x2v)
        pltpu.sync_copy(row2h.at[pl.ds(wid * NCH, NCH)], rowc)
        pltpu.sync_copy(col2h.at[pl.ds(wid * NCH, NCH)], colc)

        o16 = jnp.ones((_L,), jnp.float32)
        z16 = jnp.zeros((_L,), jnp.float32)

        def orow(r, carry):
            ones[r, pl.ds(0, _L)] = o16
            return carry

        lax.fori_loop(0, _SUB, orow, 0)

        def zrow(r, carry):
            zb[r, pl.ds(0, _L)] = z16
            return carry

        lax.fori_loop(0, STR, zrow, 0)
        pltpu.sync_copy(zb, sh_cnt.at[pl.ds(s * STR, STR)])
        plsc.subcore_barrier()

        def cnt_desc(i):
            return pltpu.make_async_copy(ones, sh_cnt.at[rowc.at[i]], sem_s)

        def chunk(i, carry):
            for t in range(NIN):
                g = i * NIN + t
                r = rowc[i, pl.ds(t * _L, _L)]
                q = colc[i, pl.ds(t * _L, _L)]
                dx = plsc.load_gather(x0v, [r]) - plsc.load_gather(x0v, [q])
                dy = plsc.load_gather(x1v, [r]) - plsc.load_gather(x1v, [q])
                dz = plsc.load_gather(x2v, [r]) - plsc.load_gather(x2v, [q])
                d2v[pl.ds(g * _L, _L)] = dx * dx + dy * dy + dz * dz
            pltpu.async_copy(ones, sh_cnt.at[rowc.at[i]], sem_s, add=True)

            @pl.when(i >= 2)
            def _():
                cnt_desc(i - 2).wait()

            return carry

        lax.fori_loop(0, NCH, chunk, 0)
        cnt_desc(NCH - 2).wait()
        cnt_desc(NCH - 1).wait()
        pltpu.sync_copy(d2v, d2h.at[pl.ds(base, EW)])
        plsc.subcore_barrier()
        pltpu.sync_copy(sh_cnt.at[pl.ds(s * STR, STR)],
                        cnt_h.at[c, pl.ds(s * STR, STR)])

    return k(x0, x1, x2, row2, col2)


_RB = 32


def _filter_body(d2_ref, w1c_ref, b1c_ref, w2_ref, b2_ref, o_ref):
    dist = jnp.sqrt(d2_ref[...])
    w1c = w1c_ref[...]
    b1c = b1c_ref[...]
    parts = [w1c * dist[r:r + 1, :] + b1c for r in range(_RB)]
    st = jnp.concatenate(parts, axis=1)
    st = st * lax.logistic(st)
    f = lax.dot_general(st, w2_ref[...], (((0,), (0,)), ((), ())),
                        preferred_element_type=jnp.float32)
    o_ref[...] = f + b2_ref[...]


def _tc_filter(d2m, W1, b1, W2, b2):
    R = d2m.shape[0]
    H = W1.shape[1]
    D = W2.shape[1]
    G = R // _RB
    BLK = _RB * 128
    return pl.pallas_call(
        _filter_body,
        grid=(G,),
        in_specs=[
            pl.BlockSpec((_RB, 128), lambda i: (i, 0)),
            pl.BlockSpec((H, 1), lambda i: (0, 0)),
            pl.BlockSpec((H, 1), lambda i: (0, 0)),
            pl.BlockSpec((H, D), lambda i: (0, 0)),
            pl.BlockSpec((1, D), lambda i: (0, 0)),
        ],
        out_specs=pl.BlockSpec((BLK, D), lambda i: (i, 0)),
        out_shape=jax.ShapeDtypeStruct((R * 128, D), jnp.float32),
    )(d2m, W1.reshape(H, 1), b1.reshape(H, 1), W2, b2.reshape(1, D))


_SUB = 80
_NSUB = 5
_BC = _SUB * _NSUB


def _sc_scatter(h2, f2, row2, col2):
    M = h2.shape[1]
    Dh = h2.shape[2]
    E = row2.shape[0] * row2.shape[1]
    ES = E // _NS
    NBC = ES // _BC
    STR = M // _NS
    ZR = STR // 5
    NZ = Dh // _L

    mesh = plsc.VectorSubcoreMesh(core_axis_name="c", subcore_axis_name="s")

    @functools.partial(
        pl.kernel,
        mesh=mesh,
        compiler_params=pltpu.CompilerParams(needs_layout_passes=False,
                                             use_tc_tiling_on_sc=False),
        out_type=jax.ShapeDtypeStruct((M, 2 * Dh), jnp.float32),
        scratch_types=[
            pltpu.VMEM((_NSUB, _SUB), jnp.int32),
            pltpu.VMEM((_NSUB, _SUB), jnp.int32),
            pltpu.VMEM((_NSUB, _SUB), jnp.int32),
            pltpu.VMEM((_NSUB, _SUB), jnp.int32),
            pltpu.VMEM((_NSUB, _SUB), jnp.int32),
            pltpu.VMEM((_BC, Dh), jnp.float32),
            pltpu.VMEM((_BC, Dh), jnp.float32),
            pltpu.VMEM((_BC, Dh), jnp.float32),
            pltpu.VMEM((ZR, Dh), jnp.float32),
            pltpu.VMEM_SHARED((M, Dh), jnp.float32),
            pltpu.SemaphoreType.DMA,
            pltpu.SemaphoreType.DMA,
            pltpu.SemaphoreType.DMA,
            pltpu.SemaphoreType.DMA,
            pltpu.SemaphoreType.DMA,
        ],
    )
    def k(h2h, f2h, row2h, col2h, sums_h,
          rowb0, rowb1, colb0, colb1, rsb, fb0, fb1, hb, zb, sh_sums,
          sem_in0, sem_in1, sem_gA, sem_gB, sem_s):
        c = lax.axis_index("c")
        s = lax.axis_index("s")
        hloc = h2h.at[c]
        cof = c * Dh

        def fire_in(g, rowb, colb, fb, sem):
            r0 = s * (ES // _SUB) + g * _NSUB
            e0 = s * ES + g * _BC
            pltpu.async_copy(row2h.at[pl.ds(r0, _NSUB)], rowb, sem)
            pltpu.async_copy(col2h.at[pl.ds(r0, _NSUB)], colb, sem)
            pltpu.async_copy(f2h.at[pl.ds(e0, _BC), pl.ds(cof, Dh)], fb, sem)

        def wait_in(rowb, colb, fb, sem):
            pltpu.make_async_copy(row2h.at[pl.ds(0, _NSUB)], rowb, sem).wait()
            pltpu.make_async_copy(col2h.at[pl.ds(0, _NSUB)], colb, sem).wait()
            pltpu.make_async_copy(f2h.at[pl.ds(0, _BC), pl.ds(cof, Dh)],
                                  fb, sem).wait()

        def gather_desc(colb, j, sem):
            return pltpu.make_async_copy(
                hloc.at[colb.at[j]], hb.at[pl.ds(j * _SUB, _SUB)], sem)

        def drain_scatters():
            for j in range(_NSUB):
                pltpu.make_async_copy(
                    hb.at[pl.ds(j * _SUB, _SUB)],
                    sh_sums.at[rsb.at[j]], sem_s).wait()

        def process(g, rowb, colb, fb, sem_in):
            @pl.when(g >= 1)
            def _():
                drain_scatters()

            wait_in(rowb, colb, fb, sem_in)
            gather_desc(colb, 0, sem_gA).start()
            gather_desc(colb, 1, sem_gB).start()
            for j in range(_NSUB):
                sem_g = sem_gA if j % 2 == 0 else sem_gB
                gather_desc(colb, j, sem_g).wait()
                if j + 2 < _NSUB:
                    gather_desc(colb, j + 2, sem_g).start()
                for t in range(_SUB // _L):
                    rsb[j, pl.ds(t * _L, _L)] = rowb[j, pl.ds(t * _L, _L)]

                def mrow(rr, cc):
                    r = j * _SUB + rr
                    for kk in range(NZ):
                        sl = pl.ds(kk * _L, _L)
                        hb[r, sl] = hb[r, sl] * fb[r, sl]
                    return cc

                lax.fori_loop(0, _SUB, mrow, 0)
                pltpu.async_copy(hb.at[pl.ds(j * _SUB, _SUB)],
                                 sh_sums.at[rsb.at[j]], sem_s, add=True)

            @pl.when(g + 2 < NBC)
            def _():
                fire_in(g + 2, rowb, colb, fb, sem_in)

        fire_in(0, rowb0, colb0, fb0, sem_in0)
        fire_in(1, rowb1, colb1, fb1, sem_in1)

        z16 = jnp.zeros((_L,), jnp.float32)

        def zrow(r, carry):
            for kk in range(NZ):
                zb[r, pl.ds(kk * _L, _L)] = z16
            return carry

        lax.fori_loop(0, ZR, zrow, 0)

        for t in range(STR // ZR):
            off = s * STR + t * ZR
            pltpu.sync_copy(zb, sh_sums.at[pl.ds(off, ZR)])
        plsc.subcore_barrier()

        def big(g2, carry):
            process(2 * g2, rowb0, colb0, fb0, sem_in0)
            process(2 * g2 + 1, rowb1, colb1, fb1, sem_in1)
            return carry

        lax.fori_loop(0, NBC // 2, big, 0)
        if NBC % 2:
            process(NBC - 1, rowb0, colb0, fb0, sem_in0)

        drain_scatters()
        plsc.subcore_barrier()
        pltpu.sync_copy(sh_sums.at[pl.ds(s * STR, STR)],
                        sums_h.at[pl.ds(s * STR, STR), pl.ds(cof, Dh)])

    return k(h2, f2, row2, col2)


def _update_body(h_ref, s_ref, s1_ref, c_ref, u1a, u1b, ub1r, g1r, be1r, u2,
                 ub2r, o_ref):
    cnt = jnp.maximum(c_ref[0, :, 0:1] + c_ref[1, :, 0:1], 1.0)
    inv = 1.0 / cnt
    agg = (s_ref[...] + s1_ref[...]) * inv
    z = jnp.dot(h_ref[...], u1a[...], preferred_element_type=jnp.float32)
    z = z + jnp.dot(agg, u1b[...], preferred_element_type=jnp.float32)
    z = z + ub1r[...]
    mu = jnp.mean(z, axis=-1, keepdims=True)
    zc = z - mu
    var = jnp.mean(zc * zc, axis=-1, keepdims=True)
    zn = zc * lax.rsqrt(var + 1e-5) * g1r[...] + be1r[...]
    zn = zn * lax.logistic(zn)
    o_ref[...] = jnp.dot(zn, u2[...], preferred_element_type=jnp.float32) \
        + ub2r[...]


def _tc_update(hf, sums, sums1, cnt, U1a, U1b, ub1, g1, be1, U2, ub2):
    M, D = hf.shape
    H = U1a.shape[1]
    BLK = 1000
    G = M // BLK
    return pl.pallas_call(
        _update_body,
        grid=(G,),
        in_specs=[
            pl.BlockSpec((BLK, D), lambda i: (i, 0)),
            pl.BlockSpec((BLK, D), lambda i: (i, 0)),
            pl.BlockSpec((BLK, D), lambda i: (i, 0)),
            pl.BlockSpec((2, BLK, _L), lambda i: (0, i, 0)),
            pl.BlockSpec((D, H), lambda i: (0, 0)),
            pl.BlockSpec((D, H), lambda i: (0, 0)),
            pl.BlockSpec((1, H), lambda i: (0, 0)),
            pl.BlockSpec((1, H), lambda i: (0, 0)),
            pl.BlockSpec((1, H), lambda i: (0, 0)),
            pl.BlockSpec((H, D), lambda i: (0, 0)),
            pl.BlockSpec((1, D), lambda i: (0, 0)),
        ],
        out_specs=pl.BlockSpec((BLK, D), lambda i: (i, 0)),
        out_shape=jax.ShapeDtypeStruct((M, D), jnp.float32),
    )(hf, sums, sums1, cnt, U1a, U1b, ub1.reshape(1, H), g1.reshape(1, H),
      be1.reshape(1, H), U2, ub2.reshape(1, D))


def kernel(x, h, edge_indices, batch_size,
           W1, b1, W2, b2, U1, ub1, g1, be1, U2, ub2):
    B, N, D = h.shape
    M = B * N
    Dh = D // 2
    row = edge_indices[0]
    col = edge_indices[1]
    xf = x.reshape(M, 3)
    hf = h.reshape(M, D)
    h2 = jnp.stack([hf[:, :Dh], hf[:, Dh:]])

    E = row.shape[0]
    row2 = row.reshape(E // _SUB, _SUB)
    col2 = col.reshape(E // _SUB, _SUB)
    d2, cnt2 = _sc_dist(xf[:, 0], xf[:, 1], xf[:, 2], row2, col2)
    E1 = 76800
    BLKE = _RB * 128
    parts = []
    for e0, e1 in ((0, E1), (E1, E)):
        ek = e1 - e0
        ekp = ((ek + BLKE - 1) // BLKE) * BLKE
        d2k = d2[e0:e1]
        d2mk = jnp.concatenate(
            [d2k, jnp.zeros((ekp - ek,), jnp.float32)]).reshape(ekp // 128,
                                                                128)
        f2k = _tc_filter(d2mk, W1, b1, W2, b2)
        parts.append(_sc_scatter(h2, f2k, row2[e0 // _SUB:e1 // _SUB],
                                 col2[e0 // _SUB:e1 // _SUB]))
    out = _tc_update(hf, parts[0], parts[1], cnt2, U1[:D], U1[D:],
                     ub1, g1, be1, U2, ub2)
    return out.reshape(B, N, D)
